# baseline (speedup 1.0000x reference)
import functools

import jax
import jax.numpy as jnp
from jax import lax
from jax.experimental import pallas as pl
from jax.experimental.pallas import tpu as pltpu
from jax.experimental.pallas import tpu_sc as plsc

N_USERS = 5000
N_ITEMS = 5000
DIM = 128
N_LAYERS = 3
N_EDGES = 320000
L2_COEF = 1e-4
BATCH = 1024
K_CAND = 5

NC, NS = 2, 16
NW = NC * NS
HALF = 5120
PADROWS = 128
N_ACC = HALF + PADROWS
ZROWS = HALF // NS
EPH = N_EDGES // 2
EPT = EPH // NS
CH = 128
NBUF = 2
NCH = 80
EPT_P = NCH * CH
PADE = EPT_P - EPT
UPT = BATCH // NS
IPT = BATCH * K_CAND // NS
ICH = 80


@functools.cache
def _make_fused():
  mesh = plsc.VectorSubcoreMesh(
      core_axis_name="c", subcore_axis_name="s",
      num_cores=NC, num_subcores=NS)

  gu_t = jax.ShapeDtypeStruct((BATCH, DIM), jnp.float32)
  gi_t = jax.ShapeDtypeStruct((BATCH * K_CAND, DIM), jnp.float32)

  @functools.partial(
      pl.kernel,
      out_type=[gu_t] * 4 + [gi_t] * 4,
      mesh=mesh,
      scratch_types=[
          pltpu.VMEM_SHARED((N_ACC, DIM), jnp.float32),
          pltpu.VMEM_SHARED((N_ACC, DIM), jnp.float32),
          pltpu.VMEM((NCH * CH,), jnp.int32),
          [pltpu.VMEM((CH,), jnp.int32) for _ in range(NBUF)],
          [pltpu.VMEM((CH,), jnp.int32) for _ in range(NBUF)],
          [pltpu.VMEM((CH, DIM), jnp.float32) for _ in range(NBUF)],
          [pltpu.SemaphoreType.DMA for _ in range(NBUF)],
          pltpu.SemaphoreType.DMA,
      ],
  )
  def _fused(emb_u, emb_i, epk, zin, guidx, giidx,
             gu0, gu1, gu2, gu3, gi0, gi1, gi2, gi3,
             acc_a, acc_b, pk, sidx, didx, rows, sems, semz):
    c = lax.axis_index("c")
    s = lax.axis_index("s")
    zb = s * ZROWS

    def unpack(j, b):
      def step(i, _):
        v = pk[pl.ds(j * CH + i * 16, 16)]
        sidx[b][pl.ds(i * 16, 16)] = v & 0xFFFF
        didx[b][pl.ds(i * 16, 16)] = lax.shift_right_logical(v, 16)
        return 0
      lax.fori_loop(0, CH // 16, step, 0)

    def edge_layer(layer, t_in, acc):
      d = lax.rem(layer + c, 2)
      pltpu.sync_copy(epk.at[d].at[s], pk)
      zero_dma = pltpu.async_copy(zin.at[pl.ds(zb, ZROWS)],
                                  acc.at[pl.ds(zb, ZROWS)], semz)
      for b in range(NBUF):
        unpack(b, b)
        pltpu.async_copy(t_in.at[sidx[b]], rows[b], sems[b])
      zero_dma.wait()
      plsc.subcore_barrier()

      def quad(k, _):
        for b in range(NBUF):
          j = NBUF * k + b
          pltpu.make_async_copy(t_in.at[sidx[b]], rows[b], sems[b]).wait()
          pltpu.sync_copy(rows[b], acc.at[didx[b]], add=True)
          unpack(j + NBUF, b)
          pltpu.async_copy(t_in.at[sidx[b]], rows[b], sems[b])
        return 0

      lax.fori_loop(0, NCH // NBUF - 1, quad, 0)

      for b in range(NBUF):
        pltpu.make_async_copy(t_in.at[sidx[b]], rows[b], sems[b]).wait()
        pltpu.sync_copy(rows[b], acc.at[didx[b]], add=True)

    def gather_items(src, dst):
      def start(q):
        base = s * IPT + q * ICH
        b = q % NBUF
        pltpu.sync_copy(giidx.at[pl.ds(base, ICH)],
                        didx[b].at[pl.ds(0, ICH)])
        pltpu.async_copy(src.at[didx[b].at[pl.ds(0, ICH)]],
                         rows[b].at[pl.ds(0, ICH)], sems[b])

      def drain(q):
        base = s * IPT + q * ICH
        b = q % NBUF
        pltpu.make_async_copy(src.at[didx[b].at[pl.ds(0, ICH)]],
                              rows[b].at[pl.ds(0, ICH)], sems[b]).wait()
        pltpu.sync_copy(rows[b].at[pl.ds(0, ICH)], dst.at[pl.ds(base, ICH)])

      start(0)
      start(1)
      for q in range(4):
        drain(q)
        if q + NBUF < 4:
          start(q + NBUF)

    def gather_users(src, dst):
      ub = s * UPT
      pltpu.sync_copy(guidx.at[pl.ds(ub, UPT)], didx[0].at[pl.ds(0, UPT)])
      pltpu.async_copy(src.at[didx[0].at[pl.ds(0, UPT)]],
                       rows[0].at[pl.ds(0, UPT)], sems[0]).wait()
      pltpu.sync_copy(rows[0].at[pl.ds(0, UPT)], dst.at[pl.ds(ub, UPT)])

    @pl.when(c == 0)
    def _():
      edge_layer(0, emb_u, acc_a)
      gather_items(emb_i, gi0)
    @pl.when(c == 1)
    def _():
      edge_layer(0, emb_i, acc_a)
      gather_users(emb_u, gu0)
    plsc.subcore_barrier()

    @pl.when(c == 0)
    def _():
      edge_layer(1, acc_a, acc_b)
      gather_items(acc_a, gi1)
    @pl.when(c == 1)
    def _():
      edge_layer(1, acc_a, acc_b)
      gather_users(acc_a, gu1)
    plsc.subcore_barrier()

    @pl.when(c == 0)
    def _():
      edge_layer(2, acc_b, acc_a)
      gather_users(acc_b, gu2)
    @pl.when(c == 1)
    def _():
      edge_layer(2, acc_b, acc_a)
      gather_items(acc_b, gi2)
    plsc.subcore_barrier()
    @pl.when(c == 0)
    def _():
      gather_items(acc_a, gi3)
    @pl.when(c == 1)
    def _():
      gather_users(acc_a, gu3)

  return _fused


def _finalize(gu0, gu1, gu2, gu3, gi0, gi1, gi2, gi3, label,
              tot_ref, scores_ref, rec_ref, emb_ref):
    u = 0.25 * (gu0[...] + gu1[...] + gu2[...] + gu3[...])
    reg = jnp.sum(u * u)
    cols = []
    for k in range(K_CAND):
        o = k * BATCH
        ik = 0.25 * (gi0[o:o + BATCH, :] + gi1[o:o + BATCH, :]
                     + gi2[o:o + BATCH, :] + gi3[o:o + BATCH, :])
        reg = reg + jnp.sum(ik * ik)
        cols.append(jnp.sum(u * ik, axis=1, keepdims=True))
    scores = jnp.concatenate(cols, axis=1)

    m = jnp.max(scores, axis=1, keepdims=True)
    e = jnp.exp(scores - m)
    probs = e / jnp.sum(e, axis=1, keepdims=True)

    lbl = label[...]
    iota_k = lax.broadcasted_iota(jnp.int32, (BATCH, K_CAND), 1)
    lmax = jnp.max(lbl, axis=1, keepdims=True)
    tgt = jnp.min(jnp.where(lbl == lmax, iota_k, K_CAND),
                  axis=1, keepdims=True)

    m2 = jnp.max(probs, axis=1, keepdims=True)
    logp = (probs - m2
            - jnp.log(jnp.sum(jnp.exp(probs - m2), axis=1, keepdims=True)))
    chosen = jnp.sum(jnp.where(iota_k == tgt, logp, 0.0), axis=1)
    rec = -jnp.sum(chosen) / BATCH
    emb = L2_COEF * reg * 0.5 / BATCH

    scores_ref[...] = scores
    tot_ref[...] = jnp.reshape(rec + emb, (1, 1))
    rec_ref[...] = jnp.reshape(rec, (1, 1))
    emb_ref[...] = jnp.reshape(emb, (1, 1))


_finalize_call = pl.pallas_call(
    _finalize,
    out_shape=[
        jax.ShapeDtypeStruct((1, 1), jnp.float32),
        jax.ShapeDtypeStruct((BATCH, K_CAND), jnp.float32),
        jax.ShapeDtypeStruct((1, 1), jnp.float32),
        jax.ShapeDtypeStruct((1, 1), jnp.float32),
    ],
)


def kernel(user_index, candidate_news_index, label,
           user_emb, item_emb, edge_src, edge_dst):
    esrc = edge_src.astype(jnp.int32)
    edst = edge_dst.astype(jnp.int32)
    src_a = esrc[:EPH]
    dst_a = edst[:EPH] - N_USERS
    src_b = esrc[EPH:] - N_USERS
    dst_b = edst[EPH:]

    pad_src = (jnp.arange(NS * PADE, dtype=jnp.int32) % N_USERS).reshape(
        NS, PADE)
    pad_dst = (HALF + jnp.arange(NS * PADE, dtype=jnp.int32) % PADROWS
               ).reshape(NS, PADE)

    def pack_dir(src, dst):
        src_p = jnp.concatenate([src.reshape(NS, EPT), pad_src], axis=1)
        dst_p = jnp.concatenate([dst.reshape(NS, EPT), pad_dst], axis=1)
        return src_p | (dst_p << 16)

    epk = jnp.stack([pack_dir(src_a, dst_a), pack_dir(src_b, dst_b)])
    zin = jnp.zeros((NS * ZROWS, DIM), jnp.float32)
    guidx = user_index.astype(jnp.int32)
    cand = candidate_news_index.astype(jnp.int32)
    giidx = jnp.concatenate([cand[:, k] for k in range(K_CAND)])

    outs = _make_fused()(user_emb, item_emb, epk, zin, guidx, giidx)
    (gu0, gu1, gu2, gu3, gi0, gi1, gi2, gi3) = outs

    tot, scores, rec, emb = _finalize_call(
        gu0, gu1, gu2, gu3, gi0, gi1, gi2, gi3, label)
    return (tot[0, 0], scores, rec[0, 0], emb[0, 0])


# speedup vs baseline: 1.3821x; 1.3821x over previous
import functools

import jax
import jax.numpy as jnp
from jax import lax
from jax.experimental import pallas as pl
from jax.experimental.pallas import tpu as pltpu
from jax.experimental.pallas import tpu_sc as plsc

N_USERS = 5000
N_ITEMS = 5000
DIM = 128
N_LAYERS = 3
N_EDGES = 320000
L2_COEF = 1e-4
BATCH = 1024
K_CAND = 5

NC, NS = 2, 16
NW = NC * NS
HALF = 5120
PADROWS = 128
N_ACC = HALF + PADROWS
ZROWS = HALF // NS
EPH = N_EDGES // 2
EPT = EPH // NS
CH = 128
NBUF = 4
NCH = 80
EPT_P = NCH * CH
PADE = EPT_P - EPT
UPT = BATCH // NS
IPT = BATCH * K_CAND // NS
ICH = 80


@functools.cache
def _make_fused():
  mesh = plsc.VectorSubcoreMesh(
      core_axis_name="c", subcore_axis_name="s",
      num_cores=NC, num_subcores=NS)

  half_t = jax.ShapeDtypeStruct((HALF, DIM), jnp.float32)
  gu_t = jax.ShapeDtypeStruct((BATCH, DIM), jnp.float32)
  gi_t = jax.ShapeDtypeStruct((BATCH * K_CAND, DIM), jnp.float32)

  @functools.partial(
      pl.kernel,
      out_type=[half_t] * 4 + [gu_t] * 4 + [gi_t] * 4,
      mesh=mesh,
      scratch_types=[
          pltpu.VMEM_SHARED((N_ACC, DIM), jnp.float32),
          pltpu.VMEM((NCH * CH,), jnp.int32),
          [pltpu.VMEM((CH,), jnp.int32) for _ in range(NBUF)],
          [pltpu.VMEM((CH,), jnp.int32) for _ in range(NBUF)],
          [pltpu.VMEM((CH, DIM), jnp.float32) for _ in range(NBUF)],
          [pltpu.SemaphoreType.DMA for _ in range(NBUF)],
          pltpu.SemaphoreType.DMA,
      ],
  )
  def _fused(emb_u, emb_i, epk, zin, guidx, giidx,
             tu1, ti1, tu2, ti2, gu0, gu1, gu2, gu3, gi0, gi1, gi2, gi3,
             acc, pk, sidx, didx, rows, sems, semz):
    c = lax.axis_index("c")
    s = lax.axis_index("s")
    zb = s * ZROWS

    def unpack(j, b):
      def step(i, _):
        v = pk[pl.ds(j * CH + i * 16, 16)]
        sidx[b][pl.ds(i * 16, 16)] = v & 0xFFFF
        didx[b][pl.ds(i * 16, 16)] = lax.shift_right_logical(v, 16)
        return 0
      lax.fori_loop(0, CH // 16, step, 0)

    def edge_layer(layer, t_in):
      d = lax.rem(layer + c, 2)
      pltpu.sync_copy(epk.at[d].at[s], pk)
      zero_dma = pltpu.async_copy(zin.at[pl.ds(zb, ZROWS)],
                                  acc.at[pl.ds(zb, ZROWS)], semz)
      for b in range(NBUF):
        unpack(b, b)
        pltpu.async_copy(t_in.at[sidx[b]], rows[b], sems[b])
      zero_dma.wait()
      plsc.subcore_barrier()

      def quad(k, _):
        for b in range(NBUF):
          j = NBUF * k + b
          pltpu.make_async_copy(t_in.at[sidx[b]], rows[b], sems[b]).wait()
          pltpu.sync_copy(rows[b], acc.at[didx[b]], add=True)
          unpack(j + NBUF, b)
          pltpu.async_copy(t_in.at[sidx[b]], rows[b], sems[b])
        return 0

      lax.fori_loop(0, NCH // NBUF - 1, quad, 0)

      for b in range(NBUF):
        pltpu.make_async_copy(t_in.at[sidx[b]], rows[b], sems[b]).wait()
        pltpu.sync_copy(rows[b], acc.at[didx[b]], add=True)

    def gather_items(src, dst):
      for q in range(4):
        base = s * IPT + q * ICH
        pltpu.sync_copy(giidx.at[pl.ds(base, ICH)],
                        didx[q].at[pl.ds(0, ICH)])
        pltpu.async_copy(src.at[didx[q].at[pl.ds(0, ICH)]],
                         rows[q].at[pl.ds(0, ICH)], sems[q])
      for q in range(4):
        base = s * IPT + q * ICH
        pltpu.make_async_copy(src.at[didx[q].at[pl.ds(0, ICH)]],
                              rows[q].at[pl.ds(0, ICH)], sems[q]).wait()
        pltpu.sync_copy(rows[q].at[pl.ds(0, ICH)], dst.at[pl.ds(base, ICH)])

    def gather_users(src, dst):
      ub = s * UPT
      pltpu.sync_copy(guidx.at[pl.ds(ub, UPT)], didx[0].at[pl.ds(0, UPT)])
      pltpu.async_copy(src.at[didx[0].at[pl.ds(0, UPT)]],
                       rows[0].at[pl.ds(0, UPT)], sems[0]).wait()
      pltpu.sync_copy(rows[0].at[pl.ds(0, UPT)], dst.at[pl.ds(ub, UPT)])

    def out_copy(dst):
      pltpu.sync_copy(acc.at[pl.ds(zb, ZROWS)], dst.at[pl.ds(zb, ZROWS)])

    @pl.when(c == 0)
    def _():
      edge_layer(0, emb_u)
      gather_items(emb_i, gi0)
    @pl.when(c == 1)
    def _():
      edge_layer(0, emb_i)
      gather_users(emb_u, gu0)
    plsc.subcore_barrier()
    @pl.when(c == 0)
    def _():
      out_copy(ti1)
    @pl.when(c == 1)
    def _():
      out_copy(tu1)
    plsc.subcore_barrier()

    @pl.when(c == 0)
    def _():
      edge_layer(1, ti1)
      gather_items(ti1, gi1)
    @pl.when(c == 1)
    def _():
      edge_layer(1, tu1)
      gather_users(tu1, gu1)
    plsc.subcore_barrier()
    @pl.when(c == 0)
    def _():
      out_copy(tu2)
    @pl.when(c == 1)
    def _():
      out_copy(ti2)
    plsc.subcore_barrier()

    @pl.when(c == 0)
    def _():
      edge_layer(2, tu2)
      gather_users(tu2, gu2)
    @pl.when(c == 1)
    def _():
      edge_layer(2, ti2)
      gather_items(ti2, gi2)
    plsc.subcore_barrier()
    @pl.when(c == 0)
    def _():
      gather_items(acc, gi3)
    @pl.when(c == 1)
    def _():
      gather_users(acc, gu3)

  return _fused


def _finalize(gu0, gu1, gu2, gu3, gi0, gi1, gi2, gi3, label,
              tot_ref, scores_ref, rec_ref, emb_ref):
    u = 0.25 * (gu0[...] + gu1[...] + gu2[...] + gu3[...])
    reg = jnp.sum(u * u)
    cols = []
    for k in range(K_CAND):
        o = k * BATCH
        ik = 0.25 * (gi0[o:o + BATCH, :] + gi1[o:o + BATCH, :]
                     + gi2[o:o + BATCH, :] + gi3[o:o + BATCH, :])
        reg = reg + jnp.sum(ik * ik)
        cols.append(jnp.sum(u * ik, axis=1, keepdims=True))
    scores = jnp.concatenate(cols, axis=1)

    m = jnp.max(scores, axis=1, keepdims=True)
    e = jnp.exp(scores - m)
    probs = e / jnp.sum(e, axis=1, keepdims=True)

    lbl = label[...]
    iota_k = lax.broadcasted_iota(jnp.int32, (BATCH, K_CAND), 1)
    lmax = jnp.max(lbl, axis=1, keepdims=True)
    tgt = jnp.min(jnp.where(lbl == lmax, iota_k, K_CAND),
                  axis=1, keepdims=True)

    m2 = jnp.max(probs, axis=1, keepdims=True)
    logp = (probs - m2
            - jnp.log(jnp.sum(jnp.exp(probs - m2), axis=1, keepdims=True)))
    chosen = jnp.sum(jnp.where(iota_k == tgt, logp, 0.0), axis=1)
    rec = -jnp.sum(chosen) / BATCH
    emb = L2_COEF * reg * 0.5 / BATCH

    scores_ref[...] = scores
    tot_ref[...] = jnp.reshape(rec + emb, (1, 1))
    rec_ref[...] = jnp.reshape(rec, (1, 1))
    emb_ref[...] = jnp.reshape(emb, (1, 1))


_finalize_call = pl.pallas_call(
    _finalize,
    out_shape=[
        jax.ShapeDtypeStruct((1, 1), jnp.float32),
        jax.ShapeDtypeStruct((BATCH, K_CAND), jnp.float32),
        jax.ShapeDtypeStruct((1, 1), jnp.float32),
        jax.ShapeDtypeStruct((1, 1), jnp.float32),
    ],
)


def kernel(user_index, candidate_news_index, label,
           user_emb, item_emb, edge_src, edge_dst):
    esrc = edge_src.astype(jnp.int32)
    edst = edge_dst.astype(jnp.int32)
    src_a = esrc[:EPH]
    dst_a = edst[:EPH] - N_USERS
    src_b = esrc[EPH:] - N_USERS
    dst_b = edst[EPH:]

    pad_src = (jnp.arange(NS * PADE, dtype=jnp.int32) % N_USERS).reshape(
        NS, PADE)
    pad_dst = (HALF + jnp.arange(NS * PADE, dtype=jnp.int32) % PADROWS
               ).reshape(NS, PADE)

    def pack_dir(src, dst):
        src_p = jnp.concatenate([src.reshape(NS, EPT), pad_src], axis=1)
        dst_p = jnp.concatenate([dst.reshape(NS, EPT), pad_dst], axis=1)
        return src_p | (dst_p << 16)

    epk = jnp.stack([pack_dir(src_a, dst_a), pack_dir(src_b, dst_b)])
    zin = jnp.zeros((NS * ZROWS, DIM), jnp.float32)
    guidx = user_index.astype(jnp.int32)
    cand = candidate_news_index.astype(jnp.int32)
    giidx = jnp.concatenate([cand[:, k] for k in range(K_CAND)])

    outs = _make_fused()(user_emb, item_emb, epk, zin, guidx, giidx)
    (tu1, ti1, tu2, ti2, gu0, gu1, gu2, gu3, gi0, gi1, gi2, gi3) = outs
    del tu1, ti1, tu2, ti2

    tot, scores, rec, emb = _finalize_call(
        gu0, gu1, gu2, gu3, gi0, gi1, gi2, gi3, label)
    return (tot[0, 0], scores, rec[0, 0], emb[0, 0])
